# Initial kernel scaffold; baseline (speedup 1.0000x reference)
import jax
import jax.numpy as jnp
from jax.experimental import pallas as pl


def kernel(feats, target, mask, W, b):
    raise NotImplementedError("write your pallas kernel here")


# speedup vs baseline: 4.2562x; 4.2562x over previous
import jax
import jax.numpy as jnp
from jax.experimental import pallas as pl
from jax.experimental.pallas import tpu as pltpu

SEQ = 512
BAT = 64
HID = 768
T = 32
TT = T * T
START = 30
END = 31
BS = 8
NBLK = SEQ // BS


def _crf_body(feats_ref, tgt_ref, msk_ref, wt_ref, b_ref, e_ref, s_ref,
              out_ref, part_ref, tg_ref):
    k = pl.program_id(0)
    fb = feats_ref[...].astype(jnp.bfloat16)
    scores = jnp.dot(fb, wt_ref[...], preferred_element_type=jnp.float32) + b_ref[...]
    lane = jax.lax.broadcasted_iota(jnp.int32, (BAT, TT), 1)
    tgt2 = tgt_ref[0]
    msk2 = msk_ref[0]
    part = part_ref[...]
    tg = jnp.where(k == 0, 0.0, tg_ref[0, 0])
    for i in range(BS):
        sc = jax.lax.slice(scores, (i * BAT, 0), ((i + 1) * BAT, TT))
        tcol = jax.lax.slice(tgt2, (0, i), (BAT, i + 1))
        mcol = jax.lax.slice(msk2, (0, i), (BAT, i + 1))
        oh = (lane == tcol).astype(jnp.float32)
        tg = tg + jnp.sum(sc * oh * mcol)
        pexp = jnp.dot(part, e_ref[...], preferred_element_type=jnp.float32,
                       precision=jax.lax.Precision.HIGHEST)
        cur = sc + pexp
        mrow = jnp.max(cur, axis=1, keepdims=True)
        ex = jnp.exp(cur - mrow)
        ssum = jnp.dot(ex, s_ref[...], preferred_element_type=jnp.float32,
                       precision=jax.lax.Precision.HIGHEST)
        rec = jnp.log(ssum) + mrow
        newpart = jnp.where(mcol > 0.0, rec, part)
        if i == 0:
            init = jax.lax.slice(sc, (0, START * T), (BAT, START * T + T))
            newpart = jnp.where(k == 0, init, newpart)
        part = newpart
    part_ref[...] = part
    tg_ref[0, 0] = tg

    @pl.when(k == NBLK - 1)
    def _():
        logz = jnp.sum(jax.lax.slice(part, (0, END), (BAT, END + 1)))
        out_ref[0, 0] = (logz - tg) / float(BAT)


def kernel(feats, target, mask, W, b):
    feats2 = feats.reshape(SEQ * BAT, HID)
    wt = W.T.astype(jnp.bfloat16)
    b2 = b.reshape(1, TT)
    tgt = target[..., 0].astype(jnp.int32).reshape(NBLK, BS, BAT).transpose(0, 2, 1)
    msk = mask.astype(jnp.float32).reshape(NBLK, BS, BAT).transpose(0, 2, 1)
    jj = jnp.arange(TT, dtype=jnp.int32)
    e_mat = (jj[None, :] // T == jnp.arange(T, dtype=jnp.int32)[:, None]).astype(jnp.float32)
    s_mat = (jj[:, None] % T == jnp.arange(T, dtype=jnp.int32)[None, :]).astype(jnp.float32)

    out = pl.pallas_call(
        _crf_body,
        grid=(NBLK,),
        in_specs=[
            pl.BlockSpec((BS * BAT, HID), lambda k: (k, 0)),
            pl.BlockSpec((1, BAT, BS), lambda k: (k, 0, 0)),
            pl.BlockSpec((1, BAT, BS), lambda k: (k, 0, 0)),
            pl.BlockSpec((HID, TT), lambda k: (0, 0)),
            pl.BlockSpec((1, TT), lambda k: (0, 0)),
            pl.BlockSpec((T, TT), lambda k: (0, 0)),
            pl.BlockSpec((TT, T), lambda k: (0, 0)),
        ],
        out_specs=pl.BlockSpec((1, 1), lambda k: (0, 0), memory_space=pltpu.SMEM),
        out_shape=jax.ShapeDtypeStruct((1, 1), jnp.float32),
        scratch_shapes=[
            pltpu.VMEM((BAT, T), jnp.float32),
            pltpu.SMEM((1, 1), jnp.float32),
        ],
        compiler_params=pltpu.CompilerParams(dimension_semantics=("arbitrary",)),
    )(feats2, tgt, msk, wt, b2, e_mat, s_mat)
    return out[0, 0]


# speedup vs baseline: 6.3729x; 1.4973x over previous
import jax
import jax.numpy as jnp
from jax.experimental import pallas as pl
from jax.experimental.pallas import tpu as pltpu

SEQ = 512
BAT = 64
HID = 768
T = 32
TT = T * T
START = 30
END = 31
BS = 8
NBLK = SEQ // BS


def _crf_body(feats_ref, tgt_ref, msk_ref, wt_ref, b_ref, e_ref, s_ref,
              out_ref, part_ref, tg_ref):
    k = pl.program_id(0)
    fb = feats_ref[...].astype(jnp.bfloat16)
    scores = jnp.dot(fb, wt_ref[...], preferred_element_type=jnp.float32) + b_ref[...]
    lane = jax.lax.broadcasted_iota(jnp.int32, (BAT, TT), 1)
    tgt2 = tgt_ref[0]
    msk2 = msk_ref[0]
    part = part_ref[...]
    tg = jnp.where(k == 0, 0.0, tg_ref[0, 0])
    for i in range(BS):
        sc = jax.lax.slice(scores, (i * BAT, 0), ((i + 1) * BAT, TT))
        tcol = jax.lax.slice(tgt2, (0, i), (BAT, i + 1))
        mcol = jax.lax.slice(msk2, (0, i), (BAT, i + 1))
        tg = tg + jnp.sum(jnp.where((lane == tcol) & (mcol > 0.0), sc, 0.0))
        pmax = jnp.max(part, axis=1, keepdims=True)
        pexp = jnp.dot(part - pmax, e_ref[...], preferred_element_type=jnp.float32)
        cur = sc + pexp
        mrow = jnp.max(cur, axis=1, keepdims=True)
        ex = jnp.exp(cur - mrow)
        ssum = jnp.dot(ex, s_ref[...], preferred_element_type=jnp.float32)
        rec = jnp.log(ssum) + (mrow + pmax)
        newpart = jnp.where(mcol > 0.0, rec, part)
        if i == 0:
            init = jax.lax.slice(sc, (0, START * T), (BAT, START * T + T))
            newpart = jnp.where(k == 0, init, newpart)
        part = newpart
    part_ref[...] = part
    tg_ref[0, 0] = tg

    @pl.when(k == NBLK - 1)
    def _():
        logz = jnp.sum(jax.lax.slice(part, (0, END), (BAT, END + 1)))
        out_ref[0, 0] = (logz - tg) / float(BAT)


def kernel(feats, target, mask, W, b):
    feats2 = feats.reshape(SEQ * BAT, HID)
    wt = W.T.astype(jnp.bfloat16)
    b2 = b.reshape(1, TT)
    tgt = target[..., 0].astype(jnp.int32).reshape(NBLK, BS, BAT).transpose(0, 2, 1)
    msk = mask.astype(jnp.float32).reshape(NBLK, BS, BAT).transpose(0, 2, 1)
    jj = jnp.arange(TT, dtype=jnp.int32)
    e_mat = (jj[None, :] // T == jnp.arange(T, dtype=jnp.int32)[:, None]).astype(jnp.float32)
    s_mat = (jj[:, None] % T == jnp.arange(T, dtype=jnp.int32)[None, :]).astype(jnp.float32)

    out = pl.pallas_call(
        _crf_body,
        grid=(NBLK,),
        in_specs=[
            pl.BlockSpec((BS * BAT, HID), lambda k: (k, 0)),
            pl.BlockSpec((1, BAT, BS), lambda k: (k, 0, 0)),
            pl.BlockSpec((1, BAT, BS), lambda k: (k, 0, 0)),
            pl.BlockSpec((HID, TT), lambda k: (0, 0)),
            pl.BlockSpec((1, TT), lambda k: (0, 0)),
            pl.BlockSpec((T, TT), lambda k: (0, 0)),
            pl.BlockSpec((TT, T), lambda k: (0, 0)),
        ],
        out_specs=pl.BlockSpec((1, 1), lambda k: (0, 0), memory_space=pltpu.SMEM),
        out_shape=jax.ShapeDtypeStruct((1, 1), jnp.float32),
        scratch_shapes=[
            pltpu.VMEM((BAT, T), jnp.float32),
            pltpu.SMEM((1, 1), jnp.float32),
        ],
        compiler_params=pltpu.CompilerParams(dimension_semantics=("arbitrary",)),
    )(feats2, tgt, msk, wt, b2, e_mat, s_mat)
    return out[0, 0]
